# baseline (speedup 1.0000x reference)
import numpy as np
import jax
import jax.numpy as jnp
from jax import lax
from jax.experimental import pallas as pl
from jax.experimental.pallas import tpu as pltpu
from jax.experimental.pallas import tpu_sc as plsc

N = 10000
NDUM = 112
NP = N + NDUM
E = 320000
D = 16
IN_CH = 128

NC = 2
NS = 16
TILES = NC * NS
SB = 128
NSTREAM = 80
EP = TILES * NSTREAM * SB
RPT = NSTREAM
MROWS = E // SB
MTAIL = MROWS - (TILES - 1) * RPT
SPT = NP // NS

_MESH = dict(core_axis_name="c", subcore_axis_name="s", num_cores=NC,
             num_subcores=NS)
_SC_PARAMS = pltpu.CompilerParams(use_tc_tiling_on_sc=False)


def _fill_rows(ref, value):
    def body(i, _):
        ref[i, :] = jnp.full((D,), value, jnp.float32)
        return 0
    lax.fori_loop(0, ref.shape[0], body, 0, unroll=8)


def _fill_flat(ref, value):
    L = ref.shape[0]
    v = jnp.full((16,), value, jnp.float32)
    for off in range(0, L - 15, 16):
        ref[pl.ds(off, 16)] = v
    if L % 16:
        ref[pl.ds(L - 16, 16)] = v


def _edge_loop(table_s, acc_s, gidx_v, sidx_v, rows0_v, rows1_v, g0_sem,
               g1_sem, deg=None):
    pltpu.async_copy(table_s.at[gidx_v.at[0]], rows0_v, g0_sem)

    def pair(p, _):
        jj = 2 * p
        pltpu.async_copy(table_s.at[gidx_v.at[jj + 1]], rows1_v, g1_sem)
        pltpu.make_async_copy(table_s.at[gidx_v.at[jj]], rows0_v, g0_sem).wait()
        if deg is not None:
            dn_s, de_s, ones_v, a_sem, b_sem = deg
            d1 = pltpu.async_copy(ones_v, dn_s.at[gidx_v.at[jj]], a_sem,
                                  add=True)
            d2 = pltpu.async_copy(ones_v, de_s.at[sidx_v.at[jj]], b_sem,
                                  add=True)
        pltpu.sync_copy(rows0_v, acc_s.at[sidx_v.at[jj]], add=True)
        if deg is not None:
            d1.wait()
            d2.wait()

        @pl.when(jj + 2 < NSTREAM)
        def _():
            pltpu.async_copy(table_s.at[gidx_v.at[jj + 2]], rows0_v, g0_sem)

        pltpu.make_async_copy(table_s.at[gidx_v.at[jj + 1]], rows1_v,
                              g1_sem).wait()
        if deg is not None:
            d3 = pltpu.async_copy(ones_v, dn_s.at[gidx_v.at[jj + 1]], a_sem,
                                  add=True)
            d4 = pltpu.async_copy(ones_v, de_s.at[sidx_v.at[jj + 1]], b_sem,
                                  add=True)
        pltpu.sync_copy(rows1_v, acc_s.at[sidx_v.at[jj + 1]], add=True)
        if deg is not None:
            d3.wait()
            d4.wait()
        return 0

    lax.fori_loop(0, NSTREAM // 2, pair, 0, unroll=2)


def _load_indices(h4_hbm, gax, sax, gidx_v, sidx_v, wid, g0_sem, g1_sem):
    r0 = wid * RPT

    @pl.when(wid < TILES - 1)
    def _():
        pltpu.async_copy(h4_hbm.at[pl.ds(r0, RPT), gax], gidx_v, g0_sem)
        pltpu.async_copy(h4_hbm.at[pl.ds(r0, RPT), sax], sidx_v, g1_sem)

    @pl.when(wid == TILES - 1)
    def _():
        m0 = (TILES - 1) * RPT
        pltpu.async_copy(h4_hbm.at[pl.ds(m0, MTAIL), gax],
                         gidx_v.at[pl.ds(0, MTAIL)], g0_sem)
        pltpu.async_copy(h4_hbm.at[pl.ds(m0, MTAIL), sax],
                         sidx_v.at[pl.ds(0, MTAIL)], g1_sem)
        base = lax.broadcasted_iota(jnp.int32, (16,), 0)

        def pad_row(r, _):
            for k in range(SB // 16):
                v = N + ((r * SB + 16 * k + base) % NDUM)
                gidx_v[r, pl.ds(16 * k, 16)] = v
                sidx_v[r, pl.ds(16 * k, 16)] = v
            return 0

        lax.fori_loop(MTAIL, RPT, pad_row, 0)


def _wait_indices(h4_hbm, gidx_v, sidx_v, wid, g0_sem, g1_sem):
    @pl.when(wid < TILES - 1)
    def _():
        pltpu.make_async_copy(h4_hbm.at[pl.ds(0, RPT), 0], gidx_v,
                              g0_sem).wait()
        pltpu.make_async_copy(h4_hbm.at[pl.ds(0, RPT), 0], sidx_v,
                              g1_sem).wait()

    @pl.when(wid == TILES - 1)
    def _():
        pltpu.make_async_copy(h4_hbm.at[pl.ds(0, MTAIL), 0],
                              gidx_v.at[pl.ds(0, MTAIL)], g0_sem).wait()
        pltpu.make_async_copy(h4_hbm.at[pl.ds(0, MTAIL), 0],
                              sidx_v.at[pl.ds(0, MTAIL)], g1_sem).wait()


def _sc_pass_deg_body(table_hbm, h4_hbm, out_hbm, dn_hbm, de_hbm,
                      table_s, acc_s, dn_s, de_s, gidx_v, sidx_v, rows0_v,
                      rows1_v, zrow_v, zflat_v, ones_v, g0_sem, g1_sem, a_sem,
                      b_sem):
    c = lax.axis_index("c")
    s = lax.axis_index("s")
    row0 = s * SPT
    slc = pl.ds(row0, SPT)
    wid = c * NS + s
    _load_indices(h4_hbm, 0, 1, gidx_v, sidx_v, wid, g0_sem, g1_sem)
    d_tbl = pltpu.async_copy(table_hbm.at[slc], table_s.at[slc], a_sem)
    _fill_rows(zrow_v, 0.0)
    _fill_flat(zflat_v, 0.0)
    _fill_flat(ones_v, 1.0)
    pltpu.sync_copy(zrow_v, acc_s.at[slc])
    pltpu.sync_copy(zflat_v, dn_s.at[pl.ds(row0, SPT)])
    pltpu.sync_copy(zflat_v, de_s.at[pl.ds(row0, SPT)])
    d_tbl.wait()
    _wait_indices(h4_hbm, gidx_v, sidx_v, wid, g0_sem, g1_sem)
    plsc.subcore_barrier()
    _edge_loop(table_s, acc_s, gidx_v, sidx_v, rows0_v, rows1_v, g0_sem,
               g1_sem, deg=(dn_s, de_s, ones_v, a_sem, b_sem))
    plsc.subcore_barrier()
    pltpu.sync_copy(acc_s.at[slc], out_hbm.at[c, slc])
    pltpu.sync_copy(dn_s.at[pl.ds(row0, SPT)], dn_hbm.at[c, pl.ds(row0, SPT)])
    pltpu.sync_copy(de_s.at[pl.ds(row0, SPT)], de_hbm.at[c, pl.ds(row0, SPT)])


_sc_pass_deg = pl.kernel(
    _sc_pass_deg_body,
    out_type=(jax.ShapeDtypeStruct((NC, NP, D), jnp.float32),
              jax.ShapeDtypeStruct((NC, NP), jnp.float32),
              jax.ShapeDtypeStruct((NC, NP), jnp.float32)),
    mesh=plsc.VectorSubcoreMesh(**_MESH),
    scratch_types=[
        pltpu.VMEM_SHARED((NP, D), jnp.float32),
        pltpu.VMEM_SHARED((NP, D), jnp.float32),
        pltpu.VMEM_SHARED((NP,), jnp.float32),
        pltpu.VMEM_SHARED((NP,), jnp.float32),
        pltpu.VMEM((RPT, SB), jnp.int32),
        pltpu.VMEM((RPT, SB), jnp.int32),
        pltpu.VMEM((SB, D), jnp.float32),
        pltpu.VMEM((SB, D), jnp.float32),
        pltpu.VMEM((SPT, D), jnp.float32),
        pltpu.VMEM((SPT,), jnp.float32),
        pltpu.VMEM((SB,), jnp.float32),
        pltpu.SemaphoreType.DMA,
        pltpu.SemaphoreType.DMA,
        pltpu.SemaphoreType.DMA,
        pltpu.SemaphoreType.DMA,
    ],
    compiler_params=_SC_PARAMS,
    name="hgnn_sc_pass_deg",
)


def _make_sc_pass(gax, sax, name):
    def body(table_hbm, h4_hbm, out_hbm, table_s, acc_s, gidx_v, sidx_v,
             rows0_v, rows1_v, zrow_v, g0_sem, g1_sem, a_sem):
        c = lax.axis_index("c")
        s = lax.axis_index("s")
        row0 = s * SPT
        slc = pl.ds(row0, SPT)
        wid = c * NS + s
        _load_indices(h4_hbm, gax, sax, gidx_v, sidx_v, wid, g0_sem, g1_sem)
        d_tbl = pltpu.async_copy(table_hbm.at[slc], table_s.at[slc], a_sem)
        _fill_rows(zrow_v, 0.0)
        pltpu.sync_copy(zrow_v, acc_s.at[slc])
        d_tbl.wait()
        _wait_indices(h4_hbm, gidx_v, sidx_v, wid, g0_sem, g1_sem)
        plsc.subcore_barrier()
        _edge_loop(table_s, acc_s, gidx_v, sidx_v, rows0_v, rows1_v, g0_sem,
                   g1_sem)
        plsc.subcore_barrier()
        pltpu.sync_copy(acc_s.at[slc], out_hbm.at[c, slc])

    return pl.kernel(
        body,
        out_type=jax.ShapeDtypeStruct((NC, NP, D), jnp.float32),
        mesh=plsc.VectorSubcoreMesh(**_MESH),
        scratch_types=[
            pltpu.VMEM_SHARED((NP, D), jnp.float32),
            pltpu.VMEM_SHARED((NP, D), jnp.float32),
            pltpu.VMEM((RPT, SB), jnp.int32),
            pltpu.VMEM((RPT, SB), jnp.int32),
            pltpu.VMEM((SB, D), jnp.float32),
            pltpu.VMEM((SB, D), jnp.float32),
            pltpu.VMEM((SPT, D), jnp.float32),
            pltpu.SemaphoreType.DMA,
            pltpu.SemaphoreType.DMA,
            pltpu.SemaphoreType.DMA,
        ],
        compiler_params=_SC_PARAMS,
        name=name,
    )


_sc_pass_a = _make_sc_pass(0, 1, "hgnn_sc_pass_a")
_sc_pass_b = _make_sc_pass(1, 0, "hgnn_sc_pass_b")


PK = 128
PR = NP * D // PK
PRN = N * D // PK
DR = NP // PK


def _tc1_body(x_ref, w_ref, b_ref, o_ref):
    wt = jnp.concatenate([w_ref[...]] * 8, axis=1)
    bt = jnp.concatenate([b_ref[...]] * 8, axis=1)
    y_rep = jnp.dot(x_ref[...], wt,
                    preferred_element_type=jnp.float32) + bt
    t = y_rep.reshape(PRN, 8, PK)
    grp = lax.broadcasted_iota(jnp.int32, (PRN, 8, PK), 2) // D
    sub = lax.broadcasted_iota(jnp.int32, (PRN, 8, PK), 1)
    out = jnp.sum(jnp.where(grp == sub, t, 0.0), axis=1)
    o_ref[...] = jnp.concatenate(
        [out, jnp.zeros((PR - PRN, PK), jnp.float32)], axis=0)


_tc1 = pl.pallas_call(
    _tc1_body,
    out_shape=jax.ShapeDtypeStruct((PR, PK), jnp.float32),
    name="hgnn_tc_in_proj",
)


def _deg_packed(dp_ref):
    m = lax.broadcasted_iota(jnp.int32, (PK, D * PK), 0)
    c = lax.broadcasted_iota(jnp.int32, (PK, D * PK), 1)
    sel = jnp.where(m == 8 * (c // PK) + (c % PK) // D, 1.0, 0.0)
    d = dp_ref[0] + dp_ref[1]
    r = jnp.where(d > 0, 1.0 / d, 0.0)
    b = jnp.dot(r, sel, preferred_element_type=jnp.float32)
    return b.reshape(DR, D, PK).reshape(PR, PK)


def _tc_norm_body(pp_ref, dep_ref, o_ref):
    o_ref[...] = (pp_ref[0] + pp_ref[1]) * _deg_packed(dep_ref)


_tc_norm = pl.pallas_call(
    _tc_norm_body,
    out_shape=jax.ShapeDtypeStruct((PR, PK), jnp.float32),
    name="hgnn_tc_norm",
)


def _tc3_body(xnp_ref, dnp_ref, w_ref, b_ref, o_ref):
    wrep = jnp.concatenate([jnp.concatenate([w_ref[...]] * 8, axis=0)] * 8,
                           axis=1)
    bi = lax.broadcasted_iota(jnp.int32, (PK, PK), 0) // D
    bj = lax.broadcasted_iota(jnp.int32, (PK, PK), 1) // D
    wb = jnp.where(bi == bj, wrep, 0.0)
    bt = jnp.concatenate([b_ref[...]] * 8, axis=1)
    h = (xnp_ref[0] + xnp_ref[1]) * _deg_packed(dnp_ref)
    h = jnp.where(h >= 0, h, 0.01 * h)
    y = jnp.dot(h, wb, preferred_element_type=jnp.float32) + bt
    rows = lax.broadcasted_iota(jnp.int32, (PR, PK), 0)
    o_ref[...] = jnp.where(rows < PRN, y, 0.0)


_tc3 = pl.pallas_call(
    _tc3_body,
    out_shape=jax.ShapeDtypeStruct((PR, PK), jnp.float32),
    name="hgnn_tc_mid",
)


def _tc5_body(xnp_ref, dnp_ref, o_ref):
    z = (xnp_ref[0] + xnp_ref[1]) * _deg_packed(dnp_ref)
    gl = lax.broadcasted_iota(jnp.int32, (PK, PK), 0) // D
    gc = lax.broadcasted_iota(jnp.int32, (PK, PK), 1) // D
    gsum = jnp.where(gl == gc, 1.0, 0.0)
    e = jnp.exp(z)
    s = jnp.dot(e, gsum, preferred_element_type=jnp.float32)
    o_ref[...] = z - jnp.log(s)


_tc5 = pl.pallas_call(
    _tc5_body,
    out_shape=jax.ShapeDtypeStruct((PR, PK), jnp.float32),
    name="hgnn_tc_out",
)


def kernel(x, H, W1, b1, W2, b2):
    h4 = jnp.transpose(H.reshape(2, MROWS, SB), (1, 0, 2))

    tbl1 = _tc1(x, W1, b1.reshape(1, D)).reshape(NP, D)
    xe_p, dn_p, de_p = _sc_pass_deg(tbl1, h4)
    dn_pp = dn_p.reshape(NC, DR, PK)
    de_pp = de_p.reshape(NC, DR, PK)
    xen = _tc_norm(xe_p.reshape(NC, PR, PK), de_pp).reshape(NP, D)
    xn_p = _sc_pass_b(xen, h4)
    tbl2 = _tc3(xn_p.reshape(NC, PR, PK), dn_pp, W2,
                b2.reshape(1, D)).reshape(NP, D)
    xe2_p = _sc_pass_a(tbl2, h4)
    xen2 = _tc_norm(xe2_p.reshape(NC, PR, PK), de_pp).reshape(NP, D)
    xn2_p = _sc_pass_b(xen2, h4)
    out = _tc5(xn2_p.reshape(NC, PR, PK), dn_pp)
    return out.reshape(NP, D)[0:N]


# speedup vs baseline: 1.0011x; 1.0011x over previous
import numpy as np
import jax
import jax.numpy as jnp
from jax import lax
from jax.experimental import pallas as pl
from jax.experimental.pallas import tpu as pltpu
from jax.experimental.pallas import tpu_sc as plsc

N = 10000
NDUM = 112
NP = N + NDUM
E = 320000
D = 16
IN_CH = 128

NC = 2
NS = 16
TILES = NC * NS
SB = 128
NSTREAM = 80
EP = TILES * NSTREAM * SB
RPT = NSTREAM
MROWS = E // SB
MTAIL = MROWS - (TILES - 1) * RPT
SPT = NP // NS

_MESH = dict(core_axis_name="c", subcore_axis_name="s", num_cores=NC,
             num_subcores=NS)
_SC_PARAMS = pltpu.CompilerParams(use_tc_tiling_on_sc=False)


def _fill_rows(ref, value):
    def body(i, _):
        ref[i, :] = jnp.full((D,), value, jnp.float32)
        return 0
    lax.fori_loop(0, ref.shape[0], body, 0, unroll=8)


def _fill_flat(ref, value):
    L = ref.shape[0]
    v = jnp.full((16,), value, jnp.float32)
    for off in range(0, L - 15, 16):
        ref[pl.ds(off, 16)] = v
    if L % 16:
        ref[pl.ds(L - 16, 16)] = v


def _edge_loop(table_s, acc_s, gidx_v, sidx_v, rows0_v, rows1_v, g0_sem,
               g1_sem, deg=None):
    pltpu.async_copy(table_s.at[gidx_v.at[0]], rows0_v, g0_sem)

    def pair(p, _):
        jj = 2 * p
        pltpu.async_copy(table_s.at[gidx_v.at[jj + 1]], rows1_v, g1_sem)
        pltpu.make_async_copy(table_s.at[gidx_v.at[jj]], rows0_v, g0_sem).wait()
        if deg is not None:
            dn_s, de_s, ones_v, a_sem, b_sem = deg
            d1 = pltpu.async_copy(ones_v, dn_s.at[gidx_v.at[jj]], a_sem,
                                  add=True)
            d2 = pltpu.async_copy(ones_v, de_s.at[sidx_v.at[jj]], b_sem,
                                  add=True)
        pltpu.sync_copy(rows0_v, acc_s.at[sidx_v.at[jj]], add=True)
        if deg is not None:
            d1.wait()
            d2.wait()

        @pl.when(jj + 2 < NSTREAM)
        def _():
            pltpu.async_copy(table_s.at[gidx_v.at[jj + 2]], rows0_v, g0_sem)

        pltpu.make_async_copy(table_s.at[gidx_v.at[jj + 1]], rows1_v,
                              g1_sem).wait()
        if deg is not None:
            d3 = pltpu.async_copy(ones_v, dn_s.at[gidx_v.at[jj + 1]], a_sem,
                                  add=True)
            d4 = pltpu.async_copy(ones_v, de_s.at[sidx_v.at[jj + 1]], b_sem,
                                  add=True)
        pltpu.sync_copy(rows1_v, acc_s.at[sidx_v.at[jj + 1]], add=True)
        if deg is not None:
            d3.wait()
            d4.wait()
        return 0

    lax.fori_loop(0, NSTREAM // 2, pair, 0)


def _load_indices(h4_hbm, gax, sax, gidx_v, sidx_v, wid, g0_sem, g1_sem):
    r0 = wid * RPT

    @pl.when(wid < TILES - 1)
    def _():
        pltpu.async_copy(h4_hbm.at[pl.ds(r0, RPT), gax], gidx_v, g0_sem)
        pltpu.async_copy(h4_hbm.at[pl.ds(r0, RPT), sax], sidx_v, g1_sem)

    @pl.when(wid == TILES - 1)
    def _():
        m0 = (TILES - 1) * RPT
        pltpu.async_copy(h4_hbm.at[pl.ds(m0, MTAIL), gax],
                         gidx_v.at[pl.ds(0, MTAIL)], g0_sem)
        pltpu.async_copy(h4_hbm.at[pl.ds(m0, MTAIL), sax],
                         sidx_v.at[pl.ds(0, MTAIL)], g1_sem)
        base = lax.broadcasted_iota(jnp.int32, (16,), 0)

        def pad_row(r, _):
            for k in range(SB // 16):
                v = N + ((r * SB + 16 * k + base) % NDUM)
                gidx_v[r, pl.ds(16 * k, 16)] = v
                sidx_v[r, pl.ds(16 * k, 16)] = v
            return 0

        lax.fori_loop(MTAIL, RPT, pad_row, 0)


def _wait_indices(h4_hbm, gidx_v, sidx_v, wid, g0_sem, g1_sem):
    @pl.when(wid < TILES - 1)
    def _():
        pltpu.make_async_copy(h4_hbm.at[pl.ds(0, RPT), 0], gidx_v,
                              g0_sem).wait()
        pltpu.make_async_copy(h4_hbm.at[pl.ds(0, RPT), 0], sidx_v,
                              g1_sem).wait()

    @pl.when(wid == TILES - 1)
    def _():
        pltpu.make_async_copy(h4_hbm.at[pl.ds(0, MTAIL), 0],
                              gidx_v.at[pl.ds(0, MTAIL)], g0_sem).wait()
        pltpu.make_async_copy(h4_hbm.at[pl.ds(0, MTAIL), 0],
                              sidx_v.at[pl.ds(0, MTAIL)], g1_sem).wait()


def _sc_pass_deg_body(table_hbm, h4_hbm, out_hbm, dn_hbm, de_hbm,
                      table_s, acc_s, dn_s, de_s, gidx_v, sidx_v, rows0_v,
                      rows1_v, zrow_v, zflat_v, ones_v, g0_sem, g1_sem, a_sem,
                      b_sem):
    c = lax.axis_index("c")
    s = lax.axis_index("s")
    row0 = s * SPT
    slc = pl.ds(row0, SPT)
    wid = c * NS + s
    _load_indices(h4_hbm, 0, 1, gidx_v, sidx_v, wid, g0_sem, g1_sem)
    d_tbl = pltpu.async_copy(table_hbm.at[slc], table_s.at[slc], a_sem)
    _fill_rows(zrow_v, 0.0)
    _fill_flat(zflat_v, 0.0)
    _fill_flat(ones_v, 1.0)
    pltpu.sync_copy(zrow_v, acc_s.at[slc])
    pltpu.sync_copy(zflat_v, dn_s.at[pl.ds(row0, SPT)])
    pltpu.sync_copy(zflat_v, de_s.at[pl.ds(row0, SPT)])
    d_tbl.wait()
    _wait_indices(h4_hbm, gidx_v, sidx_v, wid, g0_sem, g1_sem)
    plsc.subcore_barrier()
    _edge_loop(table_s, acc_s, gidx_v, sidx_v, rows0_v, rows1_v, g0_sem,
               g1_sem, deg=(dn_s, de_s, ones_v, a_sem, b_sem))
    plsc.subcore_barrier()
    pltpu.sync_copy(acc_s.at[slc], out_hbm.at[c, slc])
    pltpu.sync_copy(dn_s.at[pl.ds(row0, SPT)], dn_hbm.at[c, pl.ds(row0, SPT)])
    pltpu.sync_copy(de_s.at[pl.ds(row0, SPT)], de_hbm.at[c, pl.ds(row0, SPT)])


_sc_pass_deg = pl.kernel(
    _sc_pass_deg_body,
    out_type=(jax.ShapeDtypeStruct((NC, NP, D), jnp.float32),
              jax.ShapeDtypeStruct((NC, NP), jnp.float32),
              jax.ShapeDtypeStruct((NC, NP), jnp.float32)),
    mesh=plsc.VectorSubcoreMesh(**_MESH),
    scratch_types=[
        pltpu.VMEM_SHARED((NP, D), jnp.float32),
        pltpu.VMEM_SHARED((NP, D), jnp.float32),
        pltpu.VMEM_SHARED((NP,), jnp.float32),
        pltpu.VMEM_SHARED((NP,), jnp.float32),
        pltpu.VMEM((RPT, SB), jnp.int32),
        pltpu.VMEM((RPT, SB), jnp.int32),
        pltpu.VMEM((SB, D), jnp.float32),
        pltpu.VMEM((SB, D), jnp.float32),
        pltpu.VMEM((SPT, D), jnp.float32),
        pltpu.VMEM((SPT,), jnp.float32),
        pltpu.VMEM((SB,), jnp.float32),
        pltpu.SemaphoreType.DMA,
        pltpu.SemaphoreType.DMA,
        pltpu.SemaphoreType.DMA,
        pltpu.SemaphoreType.DMA,
    ],
    compiler_params=_SC_PARAMS,
    name="hgnn_sc_pass_deg",
)


def _make_sc_pass(gax, sax, name):
    def body(table_hbm, h4_hbm, out_hbm, table_s, acc_s, gidx_v, sidx_v,
             rows0_v, rows1_v, zrow_v, g0_sem, g1_sem, a_sem):
        c = lax.axis_index("c")
        s = lax.axis_index("s")
        row0 = s * SPT
        slc = pl.ds(row0, SPT)
        wid = c * NS + s
        _load_indices(h4_hbm, gax, sax, gidx_v, sidx_v, wid, g0_sem, g1_sem)
        d_tbl = pltpu.async_copy(table_hbm.at[slc], table_s.at[slc], a_sem)
        _fill_rows(zrow_v, 0.0)
        pltpu.sync_copy(zrow_v, acc_s.at[slc])
        d_tbl.wait()
        _wait_indices(h4_hbm, gidx_v, sidx_v, wid, g0_sem, g1_sem)
        plsc.subcore_barrier()
        _edge_loop(table_s, acc_s, gidx_v, sidx_v, rows0_v, rows1_v, g0_sem,
                   g1_sem)
        plsc.subcore_barrier()
        pltpu.sync_copy(acc_s.at[slc], out_hbm.at[c, slc])

    return pl.kernel(
        body,
        out_type=jax.ShapeDtypeStruct((NC, NP, D), jnp.float32),
        mesh=plsc.VectorSubcoreMesh(**_MESH),
        scratch_types=[
            pltpu.VMEM_SHARED((NP, D), jnp.float32),
            pltpu.VMEM_SHARED((NP, D), jnp.float32),
            pltpu.VMEM((RPT, SB), jnp.int32),
            pltpu.VMEM((RPT, SB), jnp.int32),
            pltpu.VMEM((SB, D), jnp.float32),
            pltpu.VMEM((SB, D), jnp.float32),
            pltpu.VMEM((SPT, D), jnp.float32),
            pltpu.SemaphoreType.DMA,
            pltpu.SemaphoreType.DMA,
            pltpu.SemaphoreType.DMA,
        ],
        compiler_params=_SC_PARAMS,
        name=name,
    )


_sc_pass_a = _make_sc_pass(0, 1, "hgnn_sc_pass_a")
_sc_pass_b = _make_sc_pass(1, 0, "hgnn_sc_pass_b")


PK = 128
PR = NP * D // PK
PRN = N * D // PK
DR = NP // PK


def _tc1_body(x_ref, w_ref, b_ref, o_ref):
    wt = jnp.concatenate([w_ref[...]] * 8, axis=1)
    bt = jnp.concatenate([b_ref[...]] * 8, axis=1)
    y_rep = jnp.dot(x_ref[...], wt,
                    preferred_element_type=jnp.float32) + bt
    t = y_rep.reshape(PRN, 8, PK)
    grp = lax.broadcasted_iota(jnp.int32, (PRN, 8, PK), 2) // D
    sub = lax.broadcasted_iota(jnp.int32, (PRN, 8, PK), 1)
    out = jnp.sum(jnp.where(grp == sub, t, 0.0), axis=1)
    o_ref[...] = jnp.concatenate(
        [out, jnp.zeros((PR - PRN, PK), jnp.float32)], axis=0)


_tc1 = pl.pallas_call(
    _tc1_body,
    out_shape=jax.ShapeDtypeStruct((PR, PK), jnp.float32),
    name="hgnn_tc_in_proj",
)


def _deg_packed(dp_ref):
    m = lax.broadcasted_iota(jnp.int32, (PK, D * PK), 0)
    c = lax.broadcasted_iota(jnp.int32, (PK, D * PK), 1)
    sel = jnp.where(m == 8 * (c // PK) + (c % PK) // D, 1.0, 0.0)
    d = dp_ref[0] + dp_ref[1]
    r = jnp.where(d > 0, 1.0 / d, 0.0)
    b = jnp.dot(r, sel, preferred_element_type=jnp.float32)
    return b.reshape(DR, D, PK).reshape(PR, PK)


def _tc_norm_body(pp_ref, dep_ref, o_ref):
    o_ref[...] = (pp_ref[0] + pp_ref[1]) * _deg_packed(dep_ref)


_tc_norm = pl.pallas_call(
    _tc_norm_body,
    out_shape=jax.ShapeDtypeStruct((PR, PK), jnp.float32),
    name="hgnn_tc_norm",
)


def _tc3_body(xnp_ref, dnp_ref, w_ref, b_ref, o_ref):
    wrep = jnp.concatenate([jnp.concatenate([w_ref[...]] * 8, axis=0)] * 8,
                           axis=1)
    bi = lax.broadcasted_iota(jnp.int32, (PK, PK), 0) // D
    bj = lax.broadcasted_iota(jnp.int32, (PK, PK), 1) // D
    wb = jnp.where(bi == bj, wrep, 0.0)
    bt = jnp.concatenate([b_ref[...]] * 8, axis=1)
    h = (xnp_ref[0] + xnp_ref[1]) * _deg_packed(dnp_ref)
    h = jnp.where(h >= 0, h, 0.01 * h)
    y = jnp.dot(h, wb, preferred_element_type=jnp.float32) + bt
    rows = lax.broadcasted_iota(jnp.int32, (PR, PK), 0)
    o_ref[...] = jnp.where(rows < PRN, y, 0.0)


_tc3 = pl.pallas_call(
    _tc3_body,
    out_shape=jax.ShapeDtypeStruct((PR, PK), jnp.float32),
    name="hgnn_tc_mid",
)


def _tc5_body(xnp_ref, dnp_ref, o_ref):
    z = (xnp_ref[0] + xnp_ref[1]) * _deg_packed(dnp_ref)
    gl = lax.broadcasted_iota(jnp.int32, (PK, PK), 0) // D
    gc = lax.broadcasted_iota(jnp.int32, (PK, PK), 1) // D
    gsum = jnp.where(gl == gc, 1.0, 0.0)
    e = jnp.exp(z)
    s = jnp.dot(e, gsum, preferred_element_type=jnp.float32)
    o_ref[...] = z - jnp.log(s)


_tc5 = pl.pallas_call(
    _tc5_body,
    out_shape=jax.ShapeDtypeStruct((PR, PK), jnp.float32),
    name="hgnn_tc_out",
)


def kernel(x, H, W1, b1, W2, b2):
    h4 = jnp.transpose(H.reshape(2, MROWS, SB), (1, 0, 2))

    tbl1 = _tc1(x, W1, b1.reshape(1, D)).reshape(NP, D)
    xe_p, dn_p, de_p = _sc_pass_deg(tbl1, h4)
    dn_pp = dn_p.reshape(NC, DR, PK)
    de_pp = de_p.reshape(NC, DR, PK)
    xen = _tc_norm(xe_p.reshape(NC, PR, PK), de_pp).reshape(NP, D)
    xn_p = _sc_pass_b(xen, h4)
    tbl2 = _tc3(xn_p.reshape(NC, PR, PK), dn_pp, W2,
                b2.reshape(1, D)).reshape(NP, D)
    xe2_p = _sc_pass_a(tbl2, h4)
    xen2 = _tc_norm(xe2_p.reshape(NC, PR, PK), de_pp).reshape(NP, D)
    xn2_p = _sc_pass_b(xen2, h4)
    out = _tc5(xn2_p.reshape(NC, PR, PK), dn_pp)
    return out.reshape(NP, D)[0:N]


# speedup vs baseline: 1.0019x; 1.0008x over previous
import jax
import jax.numpy as jnp
from jax import lax
from jax.experimental import pallas as pl
from jax.experimental.pallas import tpu as pltpu
from jax.experimental.pallas import tpu_sc as plsc

N = 10000
NDUM = 112
NP = N + NDUM
E = 320000
D = 16
IN_CH = 128

NC = 2
NS = 16
TILES = NC * NS
SB = 128
NSTREAM = 80
EP = TILES * NSTREAM * SB
RPT = NSTREAM
MROWS = E // SB
MTAIL = MROWS - (TILES - 1) * RPT
SPT = NP // NS

_MESH = dict(core_axis_name="c", subcore_axis_name="s", num_cores=NC,
             num_subcores=NS)
_SC_PARAMS = pltpu.CompilerParams(use_tc_tiling_on_sc=False)


def _fill_rows(ref, value):
    def body(i, _):
        ref[i, :] = jnp.full((D,), value, jnp.float32)
        return 0
    lax.fori_loop(0, ref.shape[0], body, 0, unroll=8)


def _fill_flat(ref, value):
    L = ref.shape[0]
    v = jnp.full((16,), value, jnp.float32)
    for off in range(0, L - 15, 16):
        ref[pl.ds(off, 16)] = v
    if L % 16:
        ref[pl.ds(L - 16, 16)] = v


def _edge_loop(table_s, acc_s, gidx_v, sidx_v, rows0_v, rows1_v, g0_sem,
               g1_sem, deg=None):
    pltpu.async_copy(table_s.at[gidx_v.at[0]], rows0_v, g0_sem)

    def pair(p, _):
        jj = 2 * p
        pltpu.async_copy(table_s.at[gidx_v.at[jj + 1]], rows1_v, g1_sem)
        pltpu.make_async_copy(table_s.at[gidx_v.at[jj]], rows0_v, g0_sem).wait()
        if deg is not None:
            dn_s, de_s, ones_v, a_sem, b_sem = deg
            d1 = pltpu.async_copy(ones_v, dn_s.at[gidx_v.at[jj]], a_sem,
                                  add=True)
            d2 = pltpu.async_copy(ones_v, de_s.at[sidx_v.at[jj]], b_sem,
                                  add=True)
        pltpu.sync_copy(rows0_v, acc_s.at[sidx_v.at[jj]], add=True)
        if deg is not None:
            d1.wait()
            d2.wait()

        @pl.when(jj + 2 < NSTREAM)
        def _():
            pltpu.async_copy(table_s.at[gidx_v.at[jj + 2]], rows0_v, g0_sem)

        pltpu.make_async_copy(table_s.at[gidx_v.at[jj + 1]], rows1_v,
                              g1_sem).wait()
        if deg is not None:
            d3 = pltpu.async_copy(ones_v, dn_s.at[gidx_v.at[jj + 1]], a_sem,
                                  add=True)
            d4 = pltpu.async_copy(ones_v, de_s.at[sidx_v.at[jj + 1]], b_sem,
                                  add=True)
        pltpu.sync_copy(rows1_v, acc_s.at[sidx_v.at[jj + 1]], add=True)
        if deg is not None:
            d3.wait()
            d4.wait()
        return 0

    lax.fori_loop(0, NSTREAM // 2, pair, 0)


def _load_indices(h4_hbm, gax, sax, gidx_v, sidx_v, wid, g0_sem, g1_sem):
    r0 = wid * RPT

    @pl.when(wid < TILES - 1)
    def _():
        pltpu.async_copy(h4_hbm.at[pl.ds(r0, RPT), gax], gidx_v, g0_sem)
        pltpu.async_copy(h4_hbm.at[pl.ds(r0, RPT), sax], sidx_v, g1_sem)

    @pl.when(wid == TILES - 1)
    def _():
        m0 = (TILES - 1) * RPT
        pltpu.async_copy(h4_hbm.at[pl.ds(m0, MTAIL), gax],
                         gidx_v.at[pl.ds(0, MTAIL)], g0_sem)
        pltpu.async_copy(h4_hbm.at[pl.ds(m0, MTAIL), sax],
                         sidx_v.at[pl.ds(0, MTAIL)], g1_sem)
        base = lax.broadcasted_iota(jnp.int32, (16,), 0)

        def pad_row(r, _):
            for k in range(SB // 16):
                v = N + ((r * SB + 16 * k + base) % NDUM)
                gidx_v[r, pl.ds(16 * k, 16)] = v
                sidx_v[r, pl.ds(16 * k, 16)] = v
            return 0

        lax.fori_loop(MTAIL, RPT, pad_row, 0)


def _wait_indices(h4_hbm, gidx_v, sidx_v, wid, g0_sem, g1_sem):
    @pl.when(wid < TILES - 1)
    def _():
        pltpu.make_async_copy(h4_hbm.at[pl.ds(0, RPT), 0], gidx_v,
                              g0_sem).wait()
        pltpu.make_async_copy(h4_hbm.at[pl.ds(0, RPT), 0], sidx_v,
                              g1_sem).wait()

    @pl.when(wid == TILES - 1)
    def _():
        pltpu.make_async_copy(h4_hbm.at[pl.ds(0, MTAIL), 0],
                              gidx_v.at[pl.ds(0, MTAIL)], g0_sem).wait()
        pltpu.make_async_copy(h4_hbm.at[pl.ds(0, MTAIL), 0],
                              sidx_v.at[pl.ds(0, MTAIL)], g1_sem).wait()


def _sc_pass_deg_body(table_hbm, h4_hbm, out_hbm, dn_hbm, de_hbm,
                      table_s, acc_s, dn_s, de_s, gidx_v, sidx_v, rows0_v,
                      rows1_v, zrow_v, zflat_v, ones_v, g0_sem, g1_sem, a_sem,
                      b_sem):
    c = lax.axis_index("c")
    s = lax.axis_index("s")
    row0 = s * SPT
    slc = pl.ds(row0, SPT)
    wid = c * NS + s
    _load_indices(h4_hbm, 0, 1, gidx_v, sidx_v, wid, g0_sem, g1_sem)
    d_tbl = pltpu.async_copy(table_hbm.at[slc], table_s.at[slc], a_sem)
    _fill_rows(zrow_v, 0.0)
    _fill_flat(zflat_v, 0.0)
    _fill_flat(ones_v, 1.0)
    pltpu.sync_copy(zrow_v, acc_s.at[slc])
    pltpu.sync_copy(zflat_v, dn_s.at[pl.ds(row0, SPT)])
    pltpu.sync_copy(zflat_v, de_s.at[pl.ds(row0, SPT)])
    d_tbl.wait()
    _wait_indices(h4_hbm, gidx_v, sidx_v, wid, g0_sem, g1_sem)
    plsc.subcore_barrier()
    _edge_loop(table_s, acc_s, gidx_v, sidx_v, rows0_v, rows1_v, g0_sem,
               g1_sem, deg=(dn_s, de_s, ones_v, a_sem, b_sem))
    plsc.subcore_barrier()
    pltpu.sync_copy(acc_s.at[slc], out_hbm.at[c, slc])
    pltpu.sync_copy(dn_s.at[pl.ds(row0, SPT)], dn_hbm.at[c, pl.ds(row0, SPT)])
    pltpu.sync_copy(de_s.at[pl.ds(row0, SPT)], de_hbm.at[c, pl.ds(row0, SPT)])


_sc_pass_deg = pl.kernel(
    _sc_pass_deg_body,
    out_type=(jax.ShapeDtypeStruct((NC, NP, D), jnp.float32),
              jax.ShapeDtypeStruct((NC, NP), jnp.float32),
              jax.ShapeDtypeStruct((NC, NP), jnp.float32)),
    mesh=plsc.VectorSubcoreMesh(**_MESH),
    scratch_types=[
        pltpu.VMEM_SHARED((NP, D), jnp.float32),
        pltpu.VMEM_SHARED((NP, D), jnp.float32),
        pltpu.VMEM_SHARED((NP,), jnp.float32),
        pltpu.VMEM_SHARED((NP,), jnp.float32),
        pltpu.VMEM((RPT, SB), jnp.int32),
        pltpu.VMEM((RPT, SB), jnp.int32),
        pltpu.VMEM((SB, D), jnp.float32),
        pltpu.VMEM((SB, D), jnp.float32),
        pltpu.VMEM((SPT, D), jnp.float32),
        pltpu.VMEM((SPT,), jnp.float32),
        pltpu.VMEM((SB,), jnp.float32),
        pltpu.SemaphoreType.DMA,
        pltpu.SemaphoreType.DMA,
        pltpu.SemaphoreType.DMA,
        pltpu.SemaphoreType.DMA,
    ],
    compiler_params=_SC_PARAMS,
    name="hgnn_sc_pass_deg",
)


def _make_sc_pass(gax, sax, name):
    def body(table_hbm, h4_hbm, out_hbm, table_s, acc_s, gidx_v, sidx_v,
             rows0_v, rows1_v, zrow_v, g0_sem, g1_sem, a_sem):
        c = lax.axis_index("c")
        s = lax.axis_index("s")
        row0 = s * SPT
        slc = pl.ds(row0, SPT)
        wid = c * NS + s
        _load_indices(h4_hbm, gax, sax, gidx_v, sidx_v, wid, g0_sem, g1_sem)
        d_tbl = pltpu.async_copy(table_hbm.at[slc], table_s.at[slc], a_sem)
        _fill_rows(zrow_v, 0.0)
        pltpu.sync_copy(zrow_v, acc_s.at[slc])
        d_tbl.wait()
        _wait_indices(h4_hbm, gidx_v, sidx_v, wid, g0_sem, g1_sem)
        plsc.subcore_barrier()
        _edge_loop(table_s, acc_s, gidx_v, sidx_v, rows0_v, rows1_v, g0_sem,
                   g1_sem)
        plsc.subcore_barrier()
        pltpu.sync_copy(acc_s.at[slc], out_hbm.at[c, slc])

    return pl.kernel(
        body,
        out_type=jax.ShapeDtypeStruct((NC, NP, D), jnp.float32),
        mesh=plsc.VectorSubcoreMesh(**_MESH),
        scratch_types=[
            pltpu.VMEM_SHARED((NP, D), jnp.float32),
            pltpu.VMEM_SHARED((NP, D), jnp.float32),
            pltpu.VMEM((RPT, SB), jnp.int32),
            pltpu.VMEM((RPT, SB), jnp.int32),
            pltpu.VMEM((SB, D), jnp.float32),
            pltpu.VMEM((SB, D), jnp.float32),
            pltpu.VMEM((SPT, D), jnp.float32),
            pltpu.SemaphoreType.DMA,
            pltpu.SemaphoreType.DMA,
            pltpu.SemaphoreType.DMA,
        ],
        compiler_params=_SC_PARAMS,
        name=name,
    )


_sc_pass_a = _make_sc_pass(0, 1, "hgnn_sc_pass_a")
_sc_pass_b = _make_sc_pass(1, 0, "hgnn_sc_pass_b")


PK = 128
PR = NP * D // PK
PRN = N * D // PK
DR = NP // PK


def _tc1_body(x_ref, w_ref, b_ref, o_ref):
    wt = jnp.concatenate([w_ref[...]] * 8, axis=1)
    bt = jnp.concatenate([b_ref[...]] * 8, axis=1)
    y_rep = jnp.dot(x_ref[...], wt,
                    preferred_element_type=jnp.float32) + bt
    t = y_rep.reshape(PRN, 8, PK)
    grp = lax.broadcasted_iota(jnp.int32, (PRN, 8, PK), 2) // D
    sub = lax.broadcasted_iota(jnp.int32, (PRN, 8, PK), 1)
    out = jnp.sum(jnp.where(grp == sub, t, 0.0), axis=1)
    o_ref[...] = jnp.concatenate(
        [out, jnp.zeros((PR - PRN, PK), jnp.float32)], axis=0)


_tc1 = pl.pallas_call(
    _tc1_body,
    out_shape=jax.ShapeDtypeStruct((PR, PK), jnp.float32),
    name="hgnn_tc_in_proj",
)


def _deg_packed(dp_ref):
    m = lax.broadcasted_iota(jnp.int32, (PK, D * PK), 0)
    c = lax.broadcasted_iota(jnp.int32, (PK, D * PK), 1)
    sel = jnp.where(m == 8 * (c // PK) + (c % PK) // D, 1.0, 0.0)
    d = dp_ref[0] + dp_ref[1]
    r = jnp.where(d > 0, 1.0 / d, 0.0)
    b = jnp.dot(r, sel, preferred_element_type=jnp.float32)
    return b.reshape(DR, D, PK).reshape(PR, PK)


def _tc_norm_body(pp_ref, dep_ref, o_ref):
    o_ref[...] = (pp_ref[0] + pp_ref[1]) * _deg_packed(dep_ref)


_tc_norm = pl.pallas_call(
    _tc_norm_body,
    out_shape=jax.ShapeDtypeStruct((PR, PK), jnp.float32),
    name="hgnn_tc_norm",
)


def _tc3_body(xnp_ref, dnp_ref, w_ref, b_ref, o_ref):
    wrep = jnp.concatenate([jnp.concatenate([w_ref[...]] * 8, axis=0)] * 8,
                           axis=1)
    bi = lax.broadcasted_iota(jnp.int32, (PK, PK), 0) // D
    bj = lax.broadcasted_iota(jnp.int32, (PK, PK), 1) // D
    wb = jnp.where(bi == bj, wrep, 0.0)
    bt = jnp.concatenate([b_ref[...]] * 8, axis=1)
    h = (xnp_ref[0] + xnp_ref[1]) * _deg_packed(dnp_ref)
    h = jnp.where(h >= 0, h, 0.01 * h)
    y = jnp.dot(h, wb, preferred_element_type=jnp.float32) + bt
    rows = lax.broadcasted_iota(jnp.int32, (PR, PK), 0)
    o_ref[...] = jnp.where(rows < PRN, y, 0.0)


_tc3 = pl.pallas_call(
    _tc3_body,
    out_shape=jax.ShapeDtypeStruct((PR, PK), jnp.float32),
    name="hgnn_tc_mid",
)


def _tc5_body(xnp_ref, dnp_ref, o_ref):
    z = (xnp_ref[0] + xnp_ref[1]) * _deg_packed(dnp_ref)
    gl = lax.broadcasted_iota(jnp.int32, (PK, PK), 0) // D
    gc = lax.broadcasted_iota(jnp.int32, (PK, PK), 1) // D
    gsum = jnp.where(gl == gc, 1.0, 0.0)
    e = jnp.exp(z)
    s = jnp.dot(e, gsum, preferred_element_type=jnp.float32)
    o_ref[...] = z - jnp.log(s)


_tc5 = pl.pallas_call(
    _tc5_body,
    out_shape=jax.ShapeDtypeStruct((PR, PK), jnp.float32),
    name="hgnn_tc_out",
)


def kernel(x, H, W1, b1, W2, b2):
    h4 = jnp.transpose(H.reshape(2, MROWS, SB), (1, 0, 2))

    tbl1 = _tc1(x, W1, b1.reshape(1, D)).reshape(NP, D)
    xe_p, dn_p, de_p = _sc_pass_deg(tbl1, h4)
    dn_pp = dn_p.reshape(NC, DR, PK)
    de_pp = de_p.reshape(NC, DR, PK)
    xen = _tc_norm(xe_p.reshape(NC, PR, PK), de_pp).reshape(NP, D)
    xn_p = _sc_pass_b(xen, h4)
    tbl2 = _tc3(xn_p.reshape(NC, PR, PK), dn_pp, W2,
                b2.reshape(1, D)).reshape(NP, D)
    xe2_p = _sc_pass_a(tbl2, h4)
    xen2 = _tc_norm(xe2_p.reshape(NC, PR, PK), de_pp).reshape(NP, D)
    xn2_p = _sc_pass_b(xen2, h4)
    out = _tc5(xn2_p.reshape(NC, PR, PK), dn_pp)
    return out.reshape(NP, D)[0:N]
